# baseline (speedup 1.0000x reference)
import functools

import jax
import jax.numpy as jnp
from jax import lax
from jax.experimental import pallas as pl
from jax.experimental.pallas import tpu as pltpu
from jax.experimental.pallas import tpu_sc as plsc

_BATCH = 4096
_SEQ = 200
_HIDDEN = 64
_TOTAL = _BATCH * _SEQ
_NW = 32
_ROWS_PER_W = _TOTAL // _NW
_CHUNK = 128
_NCHUNK = _ROWS_PER_W // _CHUNK


def _make_lookup():
    mesh = plsc.VectorSubcoreMesh(core_axis_name="c", subcore_axis_name="s")

    @functools.partial(
        pl.kernel,
        mesh=mesh,
        out_type=jax.ShapeDtypeStruct((_TOTAL, _HIDDEN), jnp.float32),
        scratch_types=[
            pltpu.VMEM((_CHUNK,), jnp.int32),
            pltpu.VMEM((_CHUNK, _HIDDEN), jnp.float32),
            pltpu.SemaphoreType.DMA,
        ],
        compiler_params=pltpu.CompilerParams(use_tc_tiling_on_sc=False),
    )
    def lookup(table_hbm, idx_hbm, out_hbm, idx_v, rows_v, sem):
        wid = lax.axis_index("s") * 2 + lax.axis_index("c")
        base = wid * _ROWS_PER_W

        def step(g, carry):
            off = base + g * _CHUNK
            pltpu.sync_copy(idx_hbm.at[pl.ds(off, _CHUNK)], idx_v)
            pltpu.async_copy(table_hbm.at[idx_v], rows_v, sem).wait()
            pltpu.sync_copy(rows_v, out_hbm.at[pl.ds(off, _CHUNK)])
            return carry

        lax.fori_loop(0, _NCHUNK, step, 0)

    return lookup


_lookup = _make_lookup()


@jax.jit
def kernel(item_seqs, emb):
    flat = item_seqs.reshape(_TOTAL)
    out = _lookup(emb, flat)
    return out.reshape(_BATCH, _SEQ, _HIDDEN)


# speedup vs baseline: 1.1930x; 1.1930x over previous
import functools

import jax
import jax.numpy as jnp
from jax import lax
from jax.experimental import pallas as pl
from jax.experimental.pallas import tpu as pltpu
from jax.experimental.pallas import tpu_sc as plsc

_BATCH = 4096
_SEQ = 200
_HIDDEN = 64
_TOTAL = _BATCH * _SEQ
_NW = 32
_CHUNK = 128
_NCHUNK = _TOTAL // (_NW * _CHUNK)
_BLK = 50
_NBLK = _NCHUNK // _BLK
_K = 5
_PAIRS = _BLK // (2 * _K)


def _make_lookup():
    mesh = plsc.VectorSubcoreMesh(core_axis_name="c", subcore_axis_name="s")

    @functools.partial(
        pl.kernel,
        mesh=mesh,
        out_type=jax.ShapeDtypeStruct((_TOTAL, _HIDDEN), jnp.float32),
        scratch_types=[
            pltpu.VMEM((2, _BLK, _CHUNK), jnp.int32),
            pltpu.VMEM((2, _K, _CHUNK, _HIDDEN), jnp.float32),
            pltpu.SemaphoreType.DMA,
            pltpu.SemaphoreType.DMA,
            pltpu.SemaphoreType.DMA,
            pltpu.SemaphoreType.DMA,
            pltpu.SemaphoreType.DMA,
        ],
        compiler_params=pltpu.CompilerParams(use_tc_tiling_on_sc=False),
    )
    def lookup(table_hbm, idx_hbm, out_hbm, idx_v, rows_v, isem, g0s, g1s,
               w0s, w1s):
        wid = lax.axis_index("s") * 2 + lax.axis_index("c")
        chunk0 = wid * _NCHUNK
        gsems = (g0s, g1s)
        wsems = (w0s, w1s)

        pltpu.sync_copy(idx_hbm.at[pl.ds(chunk0, _BLK)], idx_v.at[0])

        for blk in range(_NBLK):
            ib = blk % 2
            if blk > 0:
                pltpu.make_async_copy(
                    idx_hbm.at[pl.ds(chunk0, _BLK)], idx_v.at[ib], isem
                ).wait()
            if blk + 1 < _NBLK:
                pltpu.async_copy(
                    idx_hbm.at[pl.ds(chunk0 + (blk + 1) * _BLK, _BLK)],
                    idx_v.at[1 - ib],
                    isem,
                )
            base = chunk0 + blk * _BLK

            def pair(tp, carry, blk=blk, ib=ib, base=base):
                for p in range(2):
                    g0 = tp * (2 * _K) + p * _K
                    gsem = gsems[p]
                    wsem = wsems[p]
                    rows_p = rows_v.at[p]

                    def drain_writebacks():
                        for b in range(_K):
                            pltpu.make_async_copy(
                                rows_p.at[b],
                                out_hbm.at[pl.ds(0, _CHUNK)],
                                wsem,
                            ).wait()

                    if blk == 0:
                        @pl.when(tp > 0)
                        def _():
                            drain_writebacks()
                    else:
                        drain_writebacks()

                    for b in range(_K):
                        pltpu.async_copy(
                            table_hbm.at[idx_v.at[ib].at[g0 + b]],
                            rows_p.at[b],
                            gsem,
                        )
                    for b in range(_K):
                        pltpu.make_async_copy(
                            table_hbm.at[pl.ds(0, _CHUNK)],
                            rows_p.at[b],
                            gsem,
                        ).wait()
                    for b in range(_K):
                        ga = base + g0 + b
                        pltpu.async_copy(
                            rows_p.at[b],
                            out_hbm.at[pl.ds(ga * _CHUNK, _CHUNK)],
                            wsem,
                        )
                return carry

            lax.fori_loop(0, _PAIRS, pair, 0)

        for p in range(2):
            for b in range(_K):
                pltpu.make_async_copy(
                    rows_v.at[p].at[b],
                    out_hbm.at[pl.ds(0, _CHUNK)],
                    wsems[p],
                ).wait()

    return lookup


_lookup = _make_lookup()


@jax.jit
def kernel(item_seqs, emb):
    flat_idx = item_seqs.reshape(_TOTAL // _CHUNK, _CHUNK)
    out = _lookup(emb, flat_idx)
    return out.reshape(_BATCH, _SEQ, _HIDDEN)


# speedup vs baseline: 1.1980x; 1.0042x over previous
import functools

import jax
import jax.numpy as jnp
from jax import lax
from jax.experimental import pallas as pl
from jax.experimental.pallas import tpu as pltpu
from jax.experimental.pallas import tpu_sc as plsc

_BATCH = 4096
_SEQ = 200
_HIDDEN = 64
_TOTAL = _BATCH * _SEQ
_NW = 32
_CHUNK = 128
_NCHUNK = _TOTAL // (_NW * _CHUNK)
_K = 5
_NSETS = _NCHUNK // _K
_PAIRS = _NSETS // 2


def _make_lookup():
    mesh = plsc.VectorSubcoreMesh(core_axis_name="c", subcore_axis_name="s")

    @functools.partial(
        pl.kernel,
        mesh=mesh,
        out_type=jax.ShapeDtypeStruct((_TOTAL, _HIDDEN), jnp.float32),
        scratch_types=[
            pltpu.VMEM((_NCHUNK, _CHUNK), jnp.int32),
            pltpu.VMEM((2, _K, _CHUNK, _HIDDEN), jnp.float32),
            pltpu.SemaphoreType.DMA,
            pltpu.SemaphoreType.DMA,
            pltpu.SemaphoreType.DMA,
            pltpu.SemaphoreType.DMA,
        ],
        compiler_params=pltpu.CompilerParams(use_tc_tiling_on_sc=False),
    )
    def lookup(table_hbm, idx_hbm, out_hbm, idx_v, rows_v, g0s, g1s, w0s,
               w1s):
        wid = lax.axis_index("s") * 2 + lax.axis_index("c")
        chunk0 = wid * _NCHUNK
        gsems = (g0s, g1s)
        wsems = (w0s, w1s)

        pltpu.sync_copy(idx_hbm.at[pl.ds(chunk0, _NCHUNK)], idx_v)

        def fire_gathers(s, p):
            for b in range(_K):
                pltpu.async_copy(
                    table_hbm.at[idx_v.at[s * _K + b]],
                    rows_v.at[p].at[b],
                    gsems[p],
                )

        def drain(sem, dst_vmem):
            if dst_vmem:
                for b in range(_K):
                    pltpu.make_async_copy(
                        table_hbm.at[pl.ds(0, _CHUNK)],
                        rows_v.at[0].at[b],
                        sem,
                    ).wait()
            else:
                for b in range(_K):
                    pltpu.make_async_copy(
                        rows_v.at[0].at[b],
                        out_hbm.at[pl.ds(0, _CHUNK)],
                        sem,
                    ).wait()

        def fire_writebacks(s, p):
            for b in range(_K):
                ga = (chunk0 + s * _K + b) * _CHUNK
                pltpu.async_copy(
                    rows_v.at[p].at[b],
                    out_hbm.at[pl.ds(ga, _CHUNK)],
                    wsems[p],
                )

        fire_gathers(0, 0)

        def pair(t, carry):
            for p in range(2):
                s = 2 * t + p
                if p == 0:
                    @pl.when(t > 0)
                    def _():
                        drain(wsems[1], False)
                else:
                    drain(wsems[0], False)
                if p == 0:
                    fire_gathers(s + 1, 1)
                else:
                    @pl.when(t < _PAIRS - 1)
                    def _():
                        fire_gathers(s + 1, 0)
                drain(gsems[p], True)
                fire_writebacks(s, p)
            return carry

        lax.fori_loop(0, _PAIRS, pair, 0)

        drain(wsems[1], False)

    return lookup


_lookup = _make_lookup()


@jax.jit
def kernel(item_seqs, emb):
    flat_idx = item_seqs.reshape(_TOTAL // _CHUNK, _CHUNK)
    out = _lookup(emb, flat_idx)
    return out.reshape(_BATCH, _SEQ, _HIDDEN)


# speedup vs baseline: 1.2257x; 1.0231x over previous
import functools

import jax
import jax.numpy as jnp
from jax import lax
from jax.experimental import pallas as pl
from jax.experimental.pallas import tpu as pltpu
from jax.experimental.pallas import tpu_sc as plsc

_BATCH = 4096
_SEQ = 200
_HIDDEN = 64
_TOTAL = _BATCH * _SEQ
_NW = 32
_CHUNK = 128
_NCHUNK = _TOTAL // (_NW * _CHUNK)
_K = 5
_NSETS = _NCHUNK // _K
_PAIRS = _NSETS // 2


def _make_lookup():
    mesh = plsc.VectorSubcoreMesh(core_axis_name="c", subcore_axis_name="s")

    @functools.partial(
        pl.kernel,
        mesh=mesh,
        out_type=jax.ShapeDtypeStruct((_TOTAL, _HIDDEN), jnp.float32),
        scratch_types=[
            pltpu.VMEM((_NCHUNK, _CHUNK), jnp.int32),
            pltpu.VMEM((2, _K, _CHUNK, _HIDDEN), jnp.float32),
            pltpu.SemaphoreType.DMA,
            pltpu.SemaphoreType.DMA,
            pltpu.SemaphoreType.DMA,
            pltpu.SemaphoreType.DMA,
        ],
        compiler_params=pltpu.CompilerParams(use_tc_tiling_on_sc=False),
    )
    def lookup(table_hbm, idx_hbm, out_hbm, idx_v, rows_v, g0s, g1s, w0s,
               w1s):
        wid = lax.axis_index("s") * 2 + lax.axis_index("c")
        chunk0 = wid * _NCHUNK
        gsems = (g0s, g1s)
        wsems = (w0s, w1s)

        pltpu.sync_copy(idx_hbm.at[pl.ds(chunk0, _NCHUNK)], idx_v)

        def fire_gathers(s, p):
            for b in range(_K):
                pltpu.async_copy(
                    table_hbm.at[idx_v.at[s * _K + b]],
                    rows_v.at[p].at[b],
                    gsems[p],
                )

        def drain(sem, dst_vmem):
            if dst_vmem:
                for b in range(_K):
                    pltpu.make_async_copy(
                        table_hbm.at[pl.ds(0, _CHUNK)],
                        rows_v.at[0].at[b],
                        sem,
                    ).wait()
            else:
                for b in range(_K):
                    pltpu.make_async_copy(
                        rows_v.at[0].at[b],
                        out_hbm.at[pl.ds(0, _CHUNK)],
                        sem,
                    ).wait()

        def fire_writebacks(s, p):
            for b in range(_K):
                ga = (chunk0 + s * _K + b) * _CHUNK
                pltpu.async_copy(
                    rows_v.at[p].at[b],
                    out_hbm.at[pl.ds(ga, _CHUNK)],
                    wsems[p],
                )

        fire_gathers(0, 0)

        def pair(t, carry):
            for p in range(2):
                s = 2 * t + p
                if p == 0:
                    @pl.when(t > 0)
                    def _():
                        drain(wsems[1], False)
                else:
                    drain(wsems[0], False)
                if p == 0:
                    fire_gathers(s + 1, 1)
                else:
                    @pl.when(t < _PAIRS - 1)
                    def _():
                        fire_gathers(s + 1, 0)
                drain(gsems[p], True)
                fire_writebacks(s, p)
            return carry

        lax.fori_loop(0, _PAIRS, pair, 0)

        drain(wsems[1], False)

    return lookup


_lookup = _make_lookup()


@jax.jit
def kernel(item_seqs, emb):
    flat_idx = jnp.transpose(item_seqs).reshape(_TOTAL // _CHUNK, _CHUNK)
    out = _lookup(emb, flat_idx)
    return jnp.transpose(out.reshape(_SEQ, _BATCH, _HIDDEN), (1, 0, 2))
